# baseline (speedup 1.0000x reference)
import functools

import jax
import jax.numpy as jnp
from jax import lax
from jax.experimental import pallas as pl
from jax.experimental.pallas import tpu as pltpu
from jax.experimental.pallas import tpu_sc as plsc

N_TOK = 8192
D = 768
E = 64
CAP = 2 * N_TOK // E
NSLOT = E * CAP
DUMP = NSLOT
BUF_ROWS = NSLOT + 8

TB = 1024
NB = N_TOK // TB

NC = 2
NS = 16
NW = NC * NS
TPW = N_TOK // NW
CHUNK = 64
KCH = TPW // CHUNK


def _router_body(x_ref, wg_ref, slot_d_ref, slot_c_ref, scale_ref, aux_ref,
                 cnt_ref, counts_ref, psum_ref):
    b = pl.program_id(0)

    @pl.when(b == 0)
    def _init():
        counts_ref[...] = jnp.zeros_like(counts_ref)
        psum_ref[...] = jnp.zeros_like(psum_ref)

    xb = x_ref[...]
    logits = jnp.dot(xb, wg_ref[...], preferred_element_type=jnp.float32)
    m = jnp.max(logits, axis=-1, keepdims=True)
    ex = jnp.exp(logits - m)
    s = jnp.sum(ex, axis=-1, keepdims=True)
    inv_s = 1.0 / s
    probs = ex * inv_s
    gate = inv_s[:, 0]

    lane = lax.broadcasted_iota(jnp.int32, (TB, E), 1)
    is_max = logits >= m
    eidx = jnp.min(jnp.where(is_max, lane, E), axis=-1)
    onehot = (lane == eidx[:, None]).astype(jnp.float32)

    SUB = 128
    ri = lax.broadcasted_iota(jnp.int32, (SUB, SUB), 0)
    ci = lax.broadcasted_iota(jnp.int32, (SUB, SUB), 1)
    tril = (ri >= ci).astype(jnp.float32)
    off = counts_ref[...]
    pieces = []
    for p in range(TB // SUB):
        cs_p = jnp.dot(tril, onehot[p * SUB:(p + 1) * SUB],
                       preferred_element_type=jnp.float32) + off
        pieces.append(cs_p)
        off = cs_p[SUB - 1:SUB, :]
    csum = jnp.concatenate(pieces, axis=0)
    pos = (jnp.sum(csum * onehot, axis=-1) - 1.0).astype(jnp.int32)

    counts_ref[...] = off
    psum_ref[...] = psum_ref[...] + jnp.sum(probs, axis=0, keepdims=True)

    keep = pos < CAP
    slot = eidx * CAP + jnp.minimum(pos, CAP - 1)
    slot_d_ref[...] = jnp.where(keep, slot, DUMP)[None, None, :]
    slot_c_ref[...] = jnp.where(keep, slot, 0)[None, None, :]
    scale_ref[...] = jnp.where(keep, gate, 0.0)[None, None, :]

    @pl.when(b == NB - 1)
    def _aux():
        frac = counts_ref[...] / float(N_TOK)
        pmean = psum_ref[...] / float(N_TOK)
        aux_ref[...] = (float(E) * jnp.sum(frac * pmean)).reshape(1, 1)
        cnt_ref[...] = counts_ref[...].astype(jnp.int32)


def _run_router(x, Wg):
    return pl.pallas_call(
        _router_body,
        grid=(NB,),
        in_specs=[
            pl.BlockSpec((TB, D), lambda b: (b, 0)),
            pl.BlockSpec((D, E), lambda b: (0, 0)),
        ],
        out_specs=[
            pl.BlockSpec((1, 1, TB), lambda b: (b, 0, 0)),
            pl.BlockSpec((1, 1, TB), lambda b: (b, 0, 0)),
            pl.BlockSpec((1, 1, TB), lambda b: (b, 0, 0)),
            pl.BlockSpec((1, 1), lambda b: (0, 0)),
            pl.BlockSpec((1, E), lambda b: (0, 0)),
        ],
        out_shape=[
            jax.ShapeDtypeStruct((NB, 1, TB), jnp.int32),
            jax.ShapeDtypeStruct((NB, 1, TB), jnp.int32),
            jax.ShapeDtypeStruct((NB, 1, TB), jnp.float32),
            jax.ShapeDtypeStruct((1, 1), jnp.float32),
            jax.ShapeDtypeStruct((1, E), jnp.int32),
        ],
        scratch_shapes=[
            pltpu.VMEM((1, E), jnp.float32),
            pltpu.VMEM((1, E), jnp.float32),
        ],
        compiler_params=pltpu.CompilerParams(
            dimension_semantics=("arbitrary",)),
    )(x, Wg)


def _dispatch_body(x_hbm, slot_hbm, buf_hbm, idx_v, rows0, rows1,
                   si0, si1, ss0, ss1):
    wid = lax.axis_index("s") * NC + lax.axis_index("c")
    base = wid * TPW
    pltpu.sync_copy(slot_hbm.at[wid], idx_v)
    rows = (rows0, rows1)
    sin = (si0, si1)
    ssc = (ss0, ss1)
    cps = {}
    scs = {}
    for j in range(KCH):
        p = j % 2
        if j >= 2:
            scs[j - 2].wait()
        cps[j] = pltpu.async_copy(
            x_hbm.at[pl.ds(base + j * CHUNK, CHUNK)], rows[p], sin[p])
        cps[j].wait()
        scs[j] = pltpu.async_copy(rows[p], buf_hbm.at[idx_v.at[j]], ssc[p])
    scs[KCH - 2].wait()
    scs[KCH - 1].wait()


def _run_dispatch(x, slot_d):
    k = functools.partial(
        pl.kernel,
        mesh=plsc.VectorSubcoreMesh(core_axis_name="c", subcore_axis_name="s"),
        out_type=jax.ShapeDtypeStruct((BUF_ROWS, D), jnp.float32),
        scratch_types=[
            pltpu.VMEM((KCH, CHUNK), jnp.int32),
            pltpu.VMEM((CHUNK, D), jnp.float32),
            pltpu.VMEM((CHUNK, D), jnp.float32),
            pltpu.SemaphoreType.DMA,
            pltpu.SemaphoreType.DMA,
            pltpu.SemaphoreType.DMA,
            pltpu.SemaphoreType.DMA,
        ],
    )(_dispatch_body)
    return k(x, slot_d)


FB = 128
NK = CAP // FB


def _ffn_compute(xb, w1_ref, b1_ref, w2_ref, b2_ref, e):
    w1 = w1_ref[0].astype(jnp.bfloat16)
    h = jnp.dot(xb.astype(jnp.bfloat16), w1,
                preferred_element_type=jnp.float32)
    h = jnp.maximum(h + b1_ref[pl.ds(e, 1), :], 0.0)
    w2 = w2_ref[0].astype(jnp.bfloat16)
    y = jnp.dot(h.astype(jnp.bfloat16), w2,
                preferred_element_type=jnp.float32)
    return y + b2_ref[pl.ds(e, 1), :]


def _ffn_body(cnt_ref, buf_ref, w1_ref, b1_ref, w2_ref, b2_ref, out_ref):
    e = pl.program_id(0)
    cnt = cnt_ref[e]

    @pl.when(cnt > FB)
    def _full():
        out_ref[...] = _ffn_compute(buf_ref[...],
                                    w1_ref, b1_ref, w2_ref, b2_ref, e)

    @pl.when(jnp.logical_and(cnt > 0, cnt <= FB))
    def _half():
        out_ref[:FB] = _ffn_compute(buf_ref[:FB],
                                    w1_ref, b1_ref, w2_ref, b2_ref, e)


def _run_ffn(cnt, buf, W1, b1, W2, b2):
    grid_spec = pltpu.PrefetchScalarGridSpec(
        num_scalar_prefetch=1,
        grid=(E,),
        in_specs=[
            pl.BlockSpec((CAP, D), lambda e, c: (e, 0)),
            pl.BlockSpec((1, D, D), lambda e, c: (e, 0, 0)),
            pl.BlockSpec((E, D), lambda e, c: (0, 0)),
            pl.BlockSpec((1, D, D), lambda e, c: (e, 0, 0)),
            pl.BlockSpec((E, D), lambda e, c: (0, 0)),
        ],
        out_specs=pl.BlockSpec((CAP, D), lambda e, c: (e, 0)),
    )
    return pl.pallas_call(
        _ffn_body,
        grid_spec=grid_spec,
        out_shape=jax.ShapeDtypeStruct((NSLOT, D), jnp.float32),
        compiler_params=pltpu.CompilerParams(
            dimension_semantics=("parallel",)),
    )(cnt, buf, W1, b1, W2, b2)


NTH = N_TOK // 2
TPWH = NTH // NW
KCHH = TPWH // CHUNK


def _make_combine_body(half):
    def body(yb_hbm, slot_hbm, y_hbm, idx_v, rows0, rows1,
             sg0, sg1, sw0, sw1):
        wid = lax.axis_index("s") * NC + lax.axis_index("c")
        base = wid * TPWH
        pltpu.sync_copy(slot_hbm.at[half, wid], idx_v)
        rows = (rows0, rows1)
        sg = (sg0, sg1)
        sw = (sw0, sw1)
        gs = {}
        ws = {}
        for j in range(KCHH):
            p = j % 2
            if j >= 2:
                ws[j - 2].wait()
            gs[j] = pltpu.async_copy(yb_hbm.at[idx_v.at[j]], rows[p], sg[p])
            gs[j].wait()
            ws[j] = pltpu.async_copy(
                rows[p], y_hbm.at[pl.ds(base + j * CHUNK, CHUNK)], sw[p])
        for j in range(max(KCHH - 2, 0), KCHH):
            ws[j].wait()
    return body


def _run_combine_half(yb, slot_c, half):
    k = functools.partial(
        pl.kernel,
        mesh=plsc.VectorSubcoreMesh(core_axis_name="c", subcore_axis_name="s"),
        out_type=jax.ShapeDtypeStruct((NTH, D), jnp.float32),
        scratch_types=[
            pltpu.VMEM((KCHH, CHUNK), jnp.int32),
            pltpu.VMEM((CHUNK, D), jnp.float32),
            pltpu.VMEM((CHUNK, D), jnp.float32),
            pltpu.SemaphoreType.DMA,
            pltpu.SemaphoreType.DMA,
            pltpu.SemaphoreType.DMA,
            pltpu.SemaphoreType.DMA,
        ],
    )(_make_combine_body(half))
    return k(yb, slot_c)


LTB = 1024
NLB = N_TOK // LTB


def _ln_body(y_ref, x_ref, scale_ref, g_ref, b_ref, out_ref):
    sc = jnp.concatenate(
        [scale_ref[i].reshape(TB, 1) for i in range(LTB // TB)], axis=0)
    y = jnp.where(sc > 0.0, y_ref[...] * sc, 0.0)
    z = y + x_ref[...]
    mu = jnp.mean(z, axis=-1, keepdims=True)
    zc = z - mu
    var = jnp.mean(zc * zc, axis=-1, keepdims=True)
    out_ref[...] = zc * lax.rsqrt(var + 1e-5) * g_ref[...] + b_ref[...]


NLBH = NTH // LTB


def _make_ln_body(with_prev):
    def body(*refs):
        if with_prev:
            y_ref, _prev, x_ref, scale_ref, g_ref, b_ref, out_ref = refs
        else:
            y_ref, x_ref, scale_ref, g_ref, b_ref, out_ref = refs
        _ln_body(y_ref, x_ref, scale_ref, g_ref, b_ref, out_ref)
    return body


def _run_ln_half(y_half, x, scale, gamma, beta, half, prev=None):
    boff = half * NLBH
    in_specs = [
        pl.BlockSpec((LTB, D), lambda b: (b, 0)),
        pl.BlockSpec((LTB, D), lambda b, _o=boff: (b + _o, 0)),
        pl.BlockSpec((LTB // TB, 1, TB), lambda b, _o=boff: (b + _o, 0, 0)),
        pl.BlockSpec((1, D), lambda b: (0, 0)),
        pl.BlockSpec((1, D), lambda b: (0, 0)),
    ]
    args = [y_half, x, scale, gamma, beta]
    aliases = {}
    if prev is not None:
        in_specs.insert(1, pl.BlockSpec(memory_space=pl.ANY))
        args.insert(1, prev)
        aliases = {1: 0}
    return pl.pallas_call(
        _make_ln_body(prev is not None),
        grid=(NLBH,),
        in_specs=in_specs,
        out_specs=pl.BlockSpec((LTB, D), lambda b, _o=boff: (b + _o, 0)),
        out_shape=jax.ShapeDtypeStruct((N_TOK, D), jnp.float32),
        input_output_aliases=aliases,
        compiler_params=pltpu.CompilerParams(
            dimension_semantics=("arbitrary",)),
    )(*args)


@jax.jit
def kernel(x, Wg, W1, b1, W2, b2, gamma, beta):
    slot_d, slot_c, scale, aux, cnt = _run_router(x, Wg)
    slot_d = slot_d.reshape(NW, KCH, CHUNK)
    slot_c4 = slot_c.reshape(2, NW, KCHH, CHUNK)

    buf = _run_dispatch(x, slot_d)

    yb = _run_ffn(cnt.reshape(E), buf, W1, b1, W2, b2)

    g2 = gamma.reshape(1, D)
    b2d = beta.reshape(1, D)
    y_a = _run_combine_half(yb, slot_c4, 0)
    y_b = _run_combine_half(yb, slot_c4, 1)
    out_a = _run_ln_half(y_a, x, scale, g2, b2d, 0)
    out = _run_ln_half(y_b, x, scale, g2, b2d, 1, prev=out_a)
    return out, aux[0, 0]


# speedup vs baseline: 1.0015x; 1.0015x over previous
import functools

import jax
import jax.numpy as jnp
from jax import lax
from jax.experimental import pallas as pl
from jax.experimental.pallas import tpu as pltpu
from jax.experimental.pallas import tpu_sc as plsc

N_TOK = 8192
D = 768
E = 64
CAP = 2 * N_TOK // E
NSLOT = E * CAP
DUMP = NSLOT
BUF_ROWS = NSLOT + 8

TB = 1024
NB = N_TOK // TB

NC = 2
NS = 16
NW = NC * NS
TPW = N_TOK // NW
CHUNK = 64
KCH = TPW // CHUNK


def _router_body(x_ref, wg_ref, slot_d_ref, slot_c_ref, scale_ref, aux_ref,
                 cnt_ref, counts_ref, psum_ref):
    b = pl.program_id(0)

    @pl.when(b == 0)
    def _init():
        counts_ref[...] = jnp.zeros_like(counts_ref)
        psum_ref[...] = jnp.zeros_like(psum_ref)

    xb = x_ref[...]
    logits = jnp.dot(xb, wg_ref[...], preferred_element_type=jnp.float32)
    m = jnp.max(logits, axis=-1, keepdims=True)
    ex = jnp.exp(logits - m)
    s = jnp.sum(ex, axis=-1, keepdims=True)
    inv_s = 1.0 / s
    probs = ex * inv_s
    gate = inv_s[:, 0]

    lane = lax.broadcasted_iota(jnp.int32, (TB, E), 1)
    is_max = logits >= m
    eidx = jnp.min(jnp.where(is_max, lane, E), axis=-1)
    onehot = (lane == eidx[:, None]).astype(jnp.float32)

    SUB = 128
    ri = lax.broadcasted_iota(jnp.int32, (SUB, SUB), 0)
    ci = lax.broadcasted_iota(jnp.int32, (SUB, SUB), 1)
    tril = (ri >= ci).astype(jnp.float32)
    off = counts_ref[...]
    pieces = []
    for p in range(TB // SUB):
        cs_p = jnp.dot(tril, onehot[p * SUB:(p + 1) * SUB],
                       preferred_element_type=jnp.float32) + off
        pieces.append(cs_p)
        off = cs_p[SUB - 1:SUB, :]
    csum = jnp.concatenate(pieces, axis=0)
    pos = (jnp.sum(csum * onehot, axis=-1) - 1.0).astype(jnp.int32)

    counts_ref[...] = off
    psum_ref[...] = psum_ref[...] + jnp.sum(probs, axis=0, keepdims=True)

    keep = pos < CAP
    slot = eidx * CAP + jnp.minimum(pos, CAP - 1)
    slot_d_ref[...] = jnp.where(keep, slot, DUMP)[None, None, :]
    slot_c_ref[...] = jnp.where(keep, slot, 0)[None, None, :]
    scale_ref[...] = jnp.where(keep, gate, 0.0)[None, None, :]

    @pl.when(b == NB - 1)
    def _aux():
        frac = counts_ref[...] / float(N_TOK)
        pmean = psum_ref[...] / float(N_TOK)
        aux_ref[...] = (float(E) * jnp.sum(frac * pmean)).reshape(1, 1)
        cnt_ref[...] = counts_ref[...].astype(jnp.int32)


def _run_router(x, Wg):
    return pl.pallas_call(
        _router_body,
        grid=(NB,),
        in_specs=[
            pl.BlockSpec((TB, D), lambda b: (b, 0)),
            pl.BlockSpec((D, E), lambda b: (0, 0)),
        ],
        out_specs=[
            pl.BlockSpec((1, 1, TB), lambda b: (b, 0, 0)),
            pl.BlockSpec((1, 1, TB), lambda b: (b, 0, 0)),
            pl.BlockSpec((1, 1, TB), lambda b: (b, 0, 0)),
            pl.BlockSpec((1, 1), lambda b: (0, 0)),
            pl.BlockSpec((1, E), lambda b: (0, 0)),
        ],
        out_shape=[
            jax.ShapeDtypeStruct((NB, 1, TB), jnp.int32),
            jax.ShapeDtypeStruct((NB, 1, TB), jnp.int32),
            jax.ShapeDtypeStruct((NB, 1, TB), jnp.float32),
            jax.ShapeDtypeStruct((1, 1), jnp.float32),
            jax.ShapeDtypeStruct((1, E), jnp.int32),
        ],
        scratch_shapes=[
            pltpu.VMEM((1, E), jnp.float32),
            pltpu.VMEM((1, E), jnp.float32),
        ],
        compiler_params=pltpu.CompilerParams(
            dimension_semantics=("arbitrary",)),
    )(x, Wg)


def _dispatch_body(x_hbm, slot_hbm, buf_hbm, idx_v, rows0, rows1,
                   si0, si1, ss0, ss1):
    wid = lax.axis_index("s") * NC + lax.axis_index("c")
    base = wid * TPW
    pltpu.sync_copy(slot_hbm.at[wid], idx_v)
    rows = (rows0, rows1)
    sin = (si0, si1)
    ssc = (ss0, ss1)
    cps = {}
    scs = {}
    for j in range(KCH):
        p = j % 2
        if j >= 2:
            scs[j - 2].wait()
        cps[j] = pltpu.async_copy(
            x_hbm.at[pl.ds(base + j * CHUNK, CHUNK)], rows[p], sin[p])
        cps[j].wait()
        scs[j] = pltpu.async_copy(rows[p], buf_hbm.at[idx_v.at[j]], ssc[p])
    scs[KCH - 2].wait()
    scs[KCH - 1].wait()


def _run_dispatch(x, slot_d):
    k = functools.partial(
        pl.kernel,
        mesh=plsc.VectorSubcoreMesh(core_axis_name="c", subcore_axis_name="s"),
        out_type=jax.ShapeDtypeStruct((BUF_ROWS, D), jnp.float32),
        scratch_types=[
            pltpu.VMEM((KCH, CHUNK), jnp.int32),
            pltpu.VMEM((CHUNK, D), jnp.float32),
            pltpu.VMEM((CHUNK, D), jnp.float32),
            pltpu.SemaphoreType.DMA,
            pltpu.SemaphoreType.DMA,
            pltpu.SemaphoreType.DMA,
            pltpu.SemaphoreType.DMA,
        ],
    )(_dispatch_body)
    return k(x, slot_d)


FB = 128
NK = CAP // FB


def _ffn_compute(xb, w1_ref, b1_ref, w2_ref, b2_ref, e):
    w1 = w1_ref[0].astype(jnp.bfloat16)
    h = jnp.dot(xb.astype(jnp.bfloat16), w1,
                preferred_element_type=jnp.float32)
    h = jnp.maximum(h + b1_ref[pl.ds(e, 1), :], 0.0)
    w2 = w2_ref[0].astype(jnp.bfloat16)
    y = jnp.dot(h.astype(jnp.bfloat16), w2,
                preferred_element_type=jnp.float32)
    return y + b2_ref[pl.ds(e, 1), :]


def _ffn_body(cnt_ref, buf_ref, w1_ref, b1_ref, w2_ref, b2_ref, out_ref):
    e = pl.program_id(0)
    cnt = cnt_ref[e]

    @pl.when(cnt > FB)
    def _full():
        out_ref[...] = _ffn_compute(buf_ref[...],
                                    w1_ref, b1_ref, w2_ref, b2_ref, e)

    @pl.when(jnp.logical_and(cnt > 0, cnt <= FB))
    def _half():
        out_ref[:FB] = _ffn_compute(buf_ref[:FB],
                                    w1_ref, b1_ref, w2_ref, b2_ref, e)


def _run_ffn(cnt, buf, W1, b1, W2, b2):
    grid_spec = pltpu.PrefetchScalarGridSpec(
        num_scalar_prefetch=1,
        grid=(E,),
        in_specs=[
            pl.BlockSpec((CAP, D), lambda e, c: (e, 0)),
            pl.BlockSpec((1, D, D), lambda e, c: (e, 0, 0)),
            pl.BlockSpec((E, D), lambda e, c: (0, 0)),
            pl.BlockSpec((1, D, D), lambda e, c: (e, 0, 0)),
            pl.BlockSpec((E, D), lambda e, c: (0, 0)),
        ],
        out_specs=pl.BlockSpec((CAP, D), lambda e, c: (e, 0)),
    )
    return pl.pallas_call(
        _ffn_body,
        grid_spec=grid_spec,
        out_shape=jax.ShapeDtypeStruct((NSLOT, D), jnp.float32),
        compiler_params=pltpu.CompilerParams(
            dimension_semantics=("parallel",)),
    )(cnt, buf, W1, b1, W2, b2)


NTH = N_TOK // 2
TPWH = NTH // NW
KCHH = TPWH // CHUNK


def _make_combine_body(half):
    def body(yb_hbm, slot_hbm, y_hbm, idx_v, rows0, rows1,
             sg0, sg1, sw0, sw1):
        wid = lax.axis_index("s") * NC + lax.axis_index("c")
        base = wid * TPWH
        pltpu.sync_copy(slot_hbm.at[half, wid], idx_v)
        rows = (rows0, rows1)
        sg = (sg0, sg1)
        sw = (sw0, sw1)
        gs = {}
        ws = {}
        for j in range(KCHH):
            p = j % 2
            if j >= 2:
                ws[j - 2].wait()
            gs[j] = pltpu.async_copy(yb_hbm.at[idx_v.at[j]], rows[p], sg[p])
            gs[j].wait()
            ws[j] = pltpu.async_copy(
                rows[p], y_hbm.at[pl.ds(base + j * CHUNK, CHUNK)], sw[p])
        for j in range(max(KCHH - 2, 0), KCHH):
            ws[j].wait()
    return body


def _run_combine_half(yb, slot_c, half):
    k = functools.partial(
        pl.kernel,
        mesh=plsc.VectorSubcoreMesh(core_axis_name="c", subcore_axis_name="s"),
        out_type=jax.ShapeDtypeStruct((NTH, D), jnp.float32),
        scratch_types=[
            pltpu.VMEM((KCHH, CHUNK), jnp.int32),
            pltpu.VMEM((CHUNK, D), jnp.float32),
            pltpu.VMEM((CHUNK, D), jnp.float32),
            pltpu.SemaphoreType.DMA,
            pltpu.SemaphoreType.DMA,
            pltpu.SemaphoreType.DMA,
            pltpu.SemaphoreType.DMA,
        ],
    )(_make_combine_body(half))
    return k(yb, slot_c)


LTB = 2048
NLB = N_TOK // LTB


def _ln_body(y_ref, x_ref, scale_ref, g_ref, b_ref, out_ref):
    sc = jnp.concatenate(
        [scale_ref[i].reshape(TB, 1) for i in range(LTB // TB)], axis=0)
    y = jnp.where(sc > 0.0, y_ref[...] * sc, 0.0)
    z = y + x_ref[...]
    mu = jnp.mean(z, axis=-1, keepdims=True)
    zc = z - mu
    var = jnp.mean(zc * zc, axis=-1, keepdims=True)
    out_ref[...] = zc * lax.rsqrt(var + 1e-5) * g_ref[...] + b_ref[...]


NLBH = NTH // LTB


def _make_ln_body(with_prev):
    def body(*refs):
        if with_prev:
            y_ref, _prev, x_ref, scale_ref, g_ref, b_ref, out_ref = refs
        else:
            y_ref, x_ref, scale_ref, g_ref, b_ref, out_ref = refs
        _ln_body(y_ref, x_ref, scale_ref, g_ref, b_ref, out_ref)
    return body


def _run_ln_half(y_half, x, scale, gamma, beta, half, prev=None):
    boff = half * NLBH
    in_specs = [
        pl.BlockSpec((LTB, D), lambda b: (b, 0)),
        pl.BlockSpec((LTB, D), lambda b, _o=boff: (b + _o, 0)),
        pl.BlockSpec((LTB // TB, 1, TB), lambda b, _o=boff: (b + _o, 0, 0)),
        pl.BlockSpec((1, D), lambda b: (0, 0)),
        pl.BlockSpec((1, D), lambda b: (0, 0)),
    ]
    args = [y_half, x, scale, gamma, beta]
    aliases = {}
    if prev is not None:
        in_specs.insert(1, pl.BlockSpec(memory_space=pl.ANY))
        args.insert(1, prev)
        aliases = {1: 0}
    return pl.pallas_call(
        _make_ln_body(prev is not None),
        grid=(NLBH,),
        in_specs=in_specs,
        out_specs=pl.BlockSpec((LTB, D), lambda b, _o=boff: (b + _o, 0)),
        out_shape=jax.ShapeDtypeStruct((N_TOK, D), jnp.float32),
        input_output_aliases=aliases,
        compiler_params=pltpu.CompilerParams(
            dimension_semantics=("arbitrary",)),
    )(*args)


@jax.jit
def kernel(x, Wg, W1, b1, W2, b2, gamma, beta):
    slot_d, slot_c, scale, aux, cnt = _run_router(x, Wg)
    slot_d = slot_d.reshape(NW, KCH, CHUNK)
    slot_c4 = slot_c.reshape(2, NW, KCHH, CHUNK)

    buf = _run_dispatch(x, slot_d)

    yb = _run_ffn(cnt.reshape(E), buf, W1, b1, W2, b2)

    g2 = gamma.reshape(1, D)
    b2d = beta.reshape(1, D)
    y_a = _run_combine_half(yb, slot_c4, 0)
    y_b = _run_combine_half(yb, slot_c4, 1)
    out_a = _run_ln_half(y_a, x, scale, g2, b2d, 0)
    out = _run_ln_half(y_b, x, scale, g2, b2d, 1, prev=out_a)
    return out, aux[0, 0]
